# Initial kernel scaffold; baseline (speedup 1.0000x reference)
import jax
import jax.numpy as jnp
from jax.experimental import pallas as pl


def kernel(x, edge_index, edge_type, V1, c1, L1, b1, V2, c2, L2, b2, V3, c3, L3, b3):
    raise NotImplementedError("write your pallas kernel here")


# speedup vs baseline: 11.5708x; 11.5708x over previous
import functools

import jax
import jax.numpy as jnp
from jax import lax
from jax.experimental import pallas as pl
from jax.experimental.pallas import tpu as pltpu
from jax.experimental.pallas import tpu_sc as plsc

_B = 128
_LANES = 16
_NS = 16
_NC = 2


def _sel(vec, i, lane):
    return jnp.sum(jnp.where(lane == i, vec, 0))


def _make_sc_agg(n, e_pad, d_aug, r_rel, n_pad, rpt):
    trash = n
    n_rounds = r_rel // _NC

    mesh = plsc.VectorSubcoreMesh(core_axis_name="c", subcore_axis_name="s")

    @functools.partial(
        pl.kernel,
        out_type=jax.ShapeDtypeStruct((r_rel, n_pad, d_aug), jnp.float32),
        mesh=mesh,
        compiler_params=pltpu.CompilerParams(use_tc_tiling_on_sc=False),
        scratch_types=[
            pltpu.VMEM((32,), jnp.int32),
            pltpu.VMEM((_B,), jnp.int32),
            pltpu.VMEM((_B,), jnp.int32),
            pltpu.VMEM((_B, d_aug), jnp.float32),
            pltpu.VMEM((_B, d_aug), jnp.float32),
            pltpu.VMEM_SHARED((n_pad, d_aug), jnp.float32),
        ],
    )
    def sc_agg(h_hbm, src_hbm, dst_hbm, offs_hbm, agg_hbm,
               offs_v, src_v, dst_v, rows_v, zero_v, agg_sp):
        c = lax.axis_index("c")
        s = lax.axis_index("s")
        lane = lax.iota(jnp.int32, _LANES)

        pltpu.sync_copy(offs_hbm, offs_v)

        zvec = jnp.zeros((_LANES,), jnp.float32)

        def zrow(i, carry):
            for j in range(d_aug // _LANES):
                zero_v[i, pl.ds(j * _LANES, _LANES)] = zvec
            return carry

        lax.fori_loop(0, _B, zrow, 0)

        for rnd in range(n_rounds):
            r = rnd * _NC + c

            row0 = s * rpt
            nfull = rpt // _B
            rem = rpt - nfull * _B
            for kz in range(nfull):
                pltpu.sync_copy(zero_v, agg_sp.at[pl.ds(row0 + kz * _B, _B)])
            if rem:
                pltpu.sync_copy(zero_v.at[pl.ds(0, rem)],
                                agg_sp.at[pl.ds(row0 + nfull * _B, rem)])
            plsc.subcore_barrier()

            ov = offs_v[pl.ds(r, _LANES)]
            off = ov[0]
            cnt = ov[1] - off
            per_tile = (cnt + _NS - 1) // _NS
            my_start = off + s * per_tile
            my_cnt = jnp.maximum(jnp.minimum(per_tile, cnt - s * per_tile), 0)
            astart = (my_start // 8) * 8
            shift = my_start - astart
            total = shift + my_cnt
            nb = (total + _B - 1) // _B

            def batch(k, carry):
                base = astart + k * _B
                pltpu.sync_copy(src_hbm.at[pl.ds(base, _B)], src_v)
                pltpu.sync_copy(dst_hbm.at[pl.ds(base, _B)], dst_v)
                for j in range(_B // _LANES):
                    pos = lane + (k * _B + j * _LANES)
                    ok = (pos >= shift) & (pos < total)
                    dv = dst_v[pl.ds(j * _LANES, _LANES)]
                    dst_v[pl.ds(j * _LANES, _LANES)] = jnp.where(ok, dv, trash)
                pltpu.sync_copy(h_hbm.at[src_v], rows_v)
                pltpu.sync_copy(rows_v, agg_sp.at[dst_v], add=True)
                return carry

            lax.fori_loop(0, nb, batch, 0)
            plsc.subcore_barrier()

            pltpu.sync_copy(agg_sp.at[pl.ds(row0, rpt)],
                            agg_hbm.at[r, pl.ds(row0, rpt)])
            plsc.subcore_barrier()

    return sc_agg


def _make_tc_dense(n, n_pad, d_in, d_out, d_aug, r_rel, nb_basis, blk,
                   act, aug_out):
    grid = n // blk
    out_w = d_aug if aug_out else d_out

    def body(h_ref, agg_ref, v_ref, c_ref, l_ref, b_ref, o_ref):
        h = h_ref[:, :d_in]
        acc = jnp.dot(h, l_ref[...], preferred_element_type=jnp.float32)
        acc = acc + b_ref[...]
        for r in range(r_rel):
            w = c_ref[r, 0] * v_ref[0]
            for bi in range(1, nb_basis):
                w = w + c_ref[r, bi] * v_ref[bi]
            deg = agg_ref[r, :, d_in:d_in + 1]
            a = agg_ref[r, :, :d_in] / jnp.maximum(deg, 1.0)
            acc = acc + jnp.dot(a, w, preferred_element_type=jnp.float32)
        if act:
            acc = jnp.maximum(acc, 0.0)
        if aug_out:
            pad = jnp.zeros((blk, d_aug - d_out - 1), jnp.float32)
            ones = jnp.ones((blk, 1), jnp.float32)
            o_ref[...] = jnp.concatenate([acc, ones, pad], axis=1)
        else:
            o_ref[...] = acc

    return pl.pallas_call(
        body,
        grid=(grid,),
        in_specs=[
            pl.BlockSpec((blk, d_aug), lambda i: (i, 0)),
            pl.BlockSpec((r_rel, blk, d_aug), lambda i: (0, i, 0)),
            pl.BlockSpec((nb_basis, d_in, d_out), lambda i: (0, 0, 0)),
            pl.BlockSpec(memory_space=pltpu.SMEM),
            pl.BlockSpec((d_in, d_out), lambda i: (0, 0)),
            pl.BlockSpec((1, d_out), lambda i: (0, 0)),
        ],
        out_specs=pl.BlockSpec((blk, out_w), lambda i: (i, 0)),
        out_shape=jax.ShapeDtypeStruct((n, out_w), jnp.float32),
    )


def kernel(x, edge_index, edge_type, V1, c1, L1, b1, V2, c2, L2, b2,
           V3, c3, L3, b3):
    n, d_in = x.shape
    e = edge_index.shape[1]
    r_rel, nb_basis = c1.shape
    d_out = L3.shape[1]
    d_aug = d_in + _LANES
    rpt = ((n + 8 + _NS * 8 - 1) // (_NS * 8)) * 8
    n_pad = rpt * _NS
    e_pad = e + 2 * _B

    src = jnp.pad(edge_index[0], (0, e_pad - e))
    dst = jnp.pad(edge_index[1], (0, e_pad - e), constant_values=n)
    offs = jnp.searchsorted(
        edge_type, jnp.arange(r_rel + 1, dtype=edge_type.dtype)
    ).astype(jnp.int32)
    offs16 = jnp.zeros((32,), jnp.int32).at[: r_rel + 1].set(offs)

    h = jnp.concatenate(
        [x, jnp.ones((n, 1), jnp.float32),
         jnp.zeros((n, d_aug - d_in - 1), jnp.float32)], axis=1)

    sc_agg = _make_sc_agg(n, e_pad, d_aug, r_rel, n_pad, rpt)
    blk = 1000 if n % 1000 == 0 else 8

    layers = [(V1, c1, L1, b1, True, True),
              (V2, c2, L2, b2, True, True),
              (V3, c3, L3, b3, False, False)]
    for V, c, L, b, act, aug_out in layers:
        agg = sc_agg(h, src, dst, offs16)
        tc = _make_tc_dense(n, n_pad, d_in, d_out, d_aug, r_rel, nb_basis,
                            blk, act, aug_out)
        h = tc(h, agg, V, c, L, b.reshape(1, -1))
    return h


# speedup vs baseline: 17.0372x; 1.4724x over previous
import functools

import jax
import jax.numpy as jnp
from jax import lax
from jax.experimental import pallas as pl
from jax.experimental.pallas import tpu as pltpu
from jax.experimental.pallas import tpu_sc as plsc

_B = 128
_LANES = 16
_NS = 16
_NC = 2


def _sel(vec, i, lane):
    return jnp.sum(jnp.where(lane == i, vec, 0))


def _make_sc_agg(n, e_pad, d_aug, r_rel, n_pad, rpt):
    trash = n
    n_rounds = r_rel // _NC

    mesh = plsc.VectorSubcoreMesh(core_axis_name="c", subcore_axis_name="s")

    @functools.partial(
        pl.kernel,
        out_type=jax.ShapeDtypeStruct((r_rel, n_pad, d_aug), jnp.float32),
        mesh=mesh,
        compiler_params=pltpu.CompilerParams(use_tc_tiling_on_sc=False),
        scratch_types=[
            pltpu.VMEM((32,), jnp.int32),
            pltpu.VMEM((2, _B), jnp.int32),
            pltpu.VMEM((2, _B), jnp.int32),
            pltpu.VMEM((_B, d_aug), jnp.float32),
            pltpu.VMEM((_B, d_aug), jnp.float32),
            pltpu.VMEM_SHARED((n_pad, d_aug), jnp.float32),
            pltpu.SemaphoreType.DMA,
            pltpu.SemaphoreType.DMA,
        ],
    )
    def sc_agg(h_hbm, ei_hbm, offs_hbm, agg_hbm,
               offs_v, eb0, eb1, rows0, rows1, agg_sp, sem0, sem1):
        c = lax.axis_index("c")
        s = lax.axis_index("s")
        lane = lax.iota(jnp.int32, _LANES)

        pltpu.sync_copy(offs_hbm, offs_v)
        zvec = jnp.zeros((_LANES,), jnp.float32)

        for rnd in range(n_rounds):
            r = rnd * _NC + c

            def zrow(i, carry):
                for j in range(d_aug // _LANES):
                    rows0[i, pl.ds(j * _LANES, _LANES)] = zvec
                return carry

            lax.fori_loop(0, _B, zrow, 0)
            row0 = s * rpt
            nfull = rpt // _B
            rem = rpt - nfull * _B
            for kz in range(nfull):
                pltpu.sync_copy(rows0, agg_sp.at[pl.ds(row0 + kz * _B, _B)])
            if rem:
                pltpu.sync_copy(rows0.at[pl.ds(0, rem)],
                                agg_sp.at[pl.ds(row0 + nfull * _B, rem)])
            plsc.subcore_barrier()

            ov = offs_v[pl.ds(r, _LANES)]
            off = ov[0]
            cnt = ov[1] - off
            per_tile = (cnt + _NS - 1) // _NS
            my_start = off + s * per_tile
            my_cnt = jnp.maximum(jnp.minimum(per_tile, cnt - s * per_tile), 0)
            astart = (my_start // 8) * 8
            shift = my_start - astart
            total = shift + my_cnt
            nb = (total + _B - 1) // _B
            nbp = jnp.maximum((nb + 1) // 2, 1)

            def stage(k, eb, rows, sem):
                base = astart + k * _B
                pltpu.sync_copy(ei_hbm.at[:, pl.ds(base, _B)], eb)
                pltpu.async_copy(h_hbm.at[eb.at[0]], rows, sem)
                for j in range(_B // _LANES):
                    pos = lane + (k * _B + j * _LANES)
                    ok = (pos >= shift) & (pos < total)
                    dv = eb[1, pl.ds(j * _LANES, _LANES)]
                    eb[1, pl.ds(j * _LANES, _LANES)] = jnp.where(ok, dv, trash)

            def finish(eb, rows, sem):
                pltpu.make_async_copy(h_hbm.at[eb.at[0]], rows, sem).wait()
                pltpu.sync_copy(rows, agg_sp.at[eb.at[1]], add=True)

            stage(0, eb0, rows0, sem0)

            def pair(i, carry):
                stage(2 * i + 1, eb1, rows1, sem1)
                finish(eb0, rows0, sem0)
                stage(2 * i + 2, eb0, rows0, sem0)
                finish(eb1, rows1, sem1)
                return carry

            lax.fori_loop(0, nbp, pair, 0)
            finish(eb0, rows0, sem0)
            plsc.subcore_barrier()

            pltpu.sync_copy(agg_sp.at[pl.ds(row0, rpt)],
                            agg_hbm.at[r, pl.ds(row0, rpt)])
            plsc.subcore_barrier()

    return sc_agg


def _make_tc_dense(n, n_pad, d_in, d_out, d_aug, r_rel, nb_basis, blk,
                   act, aug_out):
    grid = n // blk
    out_w = d_aug if aug_out else d_out

    def body(h_ref, agg_ref, v_ref, c_ref, l_ref, b_ref, o_ref):
        h = h_ref[:, :d_in]
        acc = jnp.dot(h, l_ref[...], preferred_element_type=jnp.float32)
        acc = acc + b_ref[...]
        for r in range(r_rel):
            w = c_ref[r, 0] * v_ref[0]
            for bi in range(1, nb_basis):
                w = w + c_ref[r, bi] * v_ref[bi]
            deg = agg_ref[r, :, d_in:d_in + 1]
            a = agg_ref[r, :, :d_in] / jnp.maximum(deg, 1.0)
            acc = acc + jnp.dot(a, w, preferred_element_type=jnp.float32)
        if act:
            acc = jnp.maximum(acc, 0.0)
        if aug_out:
            pad = jnp.zeros((blk, d_aug - d_out - 1), jnp.float32)
            ones = jnp.ones((blk, 1), jnp.float32)
            o_ref[...] = jnp.concatenate([acc, ones, pad], axis=1)
        else:
            o_ref[...] = acc

    return pl.pallas_call(
        body,
        grid=(grid,),
        in_specs=[
            pl.BlockSpec((blk, d_aug), lambda i: (i, 0)),
            pl.BlockSpec((r_rel, blk, d_aug), lambda i: (0, i, 0)),
            pl.BlockSpec((nb_basis, d_in, d_out), lambda i: (0, 0, 0)),
            pl.BlockSpec(memory_space=pltpu.SMEM),
            pl.BlockSpec((d_in, d_out), lambda i: (0, 0)),
            pl.BlockSpec((1, d_out), lambda i: (0, 0)),
        ],
        out_specs=pl.BlockSpec((blk, out_w), lambda i: (i, 0)),
        out_shape=jax.ShapeDtypeStruct((n, out_w), jnp.float32),
    )


def kernel(x, edge_index, edge_type, V1, c1, L1, b1, V2, c2, L2, b2,
           V3, c3, L3, b3):
    n, d_in = x.shape
    e = edge_index.shape[1]
    r_rel, nb_basis = c1.shape
    d_out = L3.shape[1]
    d_aug = d_in + _LANES
    rpt = ((n + 8 + _NS * 8 - 1) // (_NS * 8)) * 8
    n_pad = rpt * _NS
    e_pad = e + 4 * _B

    pad = jnp.stack([jnp.zeros((e_pad - e,), edge_index.dtype),
                     jnp.full((e_pad - e,), n, edge_index.dtype)])
    ei_pad = jnp.concatenate([edge_index, pad], axis=1)
    offs = jnp.searchsorted(
        edge_type, jnp.arange(r_rel + 1, dtype=edge_type.dtype)
    ).astype(jnp.int32)
    offs16 = jnp.zeros((32,), jnp.int32).at[: r_rel + 1].set(offs)

    h = jnp.concatenate(
        [x, jnp.ones((n, 1), jnp.float32),
         jnp.zeros((n, d_aug - d_in - 1), jnp.float32)], axis=1)

    sc_agg = _make_sc_agg(n, e_pad, d_aug, r_rel, n_pad, rpt)
    blk = 1000 if n % 1000 == 0 else 8

    layers = [(V1, c1, L1, b1, True, True),
              (V2, c2, L2, b2, True, True),
              (V3, c3, L3, b3, False, False)]
    for V, c, L, b, act, aug_out in layers:
        agg = sc_agg(h, ei_pad, offs16)
        tc = _make_tc_dense(n, n_pad, d_in, d_out, d_aug, r_rel, nb_basis,
                            blk, act, aug_out)
        h = tc(h, agg, V, c, L, b.reshape(1, -1))
    return h
